# baseline (speedup 1.0000x reference)
import jax
import jax.numpy as jnp
from jax.experimental import pallas as pl


def _embed_sum_block(x_ref, w_ref, o_ref):
    o_ref[...] = jnp.dot(x_ref[...], w_ref[...],
                         preferred_element_type=jnp.float32)


def kernel(inputs, embedding_matrix):
    batch, timesteps, n_codes = inputs.shape
    embed_size = embedding_matrix.shape[1]
    m = batch * timesteps
    x = inputs.reshape(m, n_codes)

    block_m = 2048 if m % 2048 == 0 else m
    grid = (m // block_m,)

    out = pl.pallas_call(
        _embed_sum_block,
        grid=grid,
        in_specs=[
            pl.BlockSpec((block_m, n_codes), lambda i: (i, 0)),
            pl.BlockSpec((n_codes, embed_size), lambda i: (0, 0)),
        ],
        out_specs=pl.BlockSpec((block_m, embed_size), lambda i: (i, 0)),
        out_shape=jax.ShapeDtypeStruct((m, embed_size), jnp.float32),
    )(x, embedding_matrix)

    return out.reshape(batch, timesteps, embed_size)


# speedup vs baseline: 1.3769x; 1.3769x over previous
import jax
import jax.numpy as jnp
from jax.experimental import pallas as pl


def _embed_sum_block(x_ref, w_ref, o_ref):
    w = w_ref[...]
    timesteps = x_ref.shape[1]
    for t in range(timesteps):
        o_ref[:, t, :] = jnp.dot(x_ref[:, t, :], w,
                                 preferred_element_type=jnp.float32)


def kernel(inputs, embedding_matrix):
    batch, timesteps, n_codes = inputs.shape
    embed_size = embedding_matrix.shape[1]

    block_b = 128 if batch % 128 == 0 else batch
    grid = (batch // block_b,)

    return pl.pallas_call(
        _embed_sum_block,
        grid=grid,
        in_specs=[
            pl.BlockSpec((block_b, timesteps, n_codes), lambda i: (i, 0, 0)),
            pl.BlockSpec((n_codes, embed_size), lambda i: (0, 0)),
        ],
        out_specs=pl.BlockSpec((block_b, timesteps, embed_size),
                               lambda i: (i, 0, 0)),
        out_shape=jax.ShapeDtypeStruct((batch, timesteps, embed_size),
                                       jnp.float32),
    )(inputs, embedding_matrix)
